# baseline (speedup 1.0000x reference)
import jax
import jax.numpy as jnp
from jax.experimental import pallas as pl
from jax.experimental.pallas import tpu as pltpu

B = 512
G = 1000
D = 128
BT = 32


def _fused_kernel(x_ref, emb_ref, feat_ref, ge_ref):
    x_blk = x_ref[...]
    e = emb_ref[...]
    ge_ref[...] = x_blk[:, :, None] * e[None, :, :]
    feat_ref[...] = jnp.dot(x_blk, e, preferred_element_type=jnp.float32)


def kernel(x_dict, emb):
    grid = (B // BT,)
    feat, gene_emb = pl.pallas_call(
        _fused_kernel,
        grid=grid,
        in_specs=[
            pl.BlockSpec((BT, G), lambda i: (i, 0)),
            pl.BlockSpec((G, D), lambda i: (0, 0)),
        ],
        out_specs=[
            pl.BlockSpec((BT, D), lambda i: (i, 0)),
            pl.BlockSpec((BT, G, D), lambda i: (i, 0, 0)),
        ],
        out_shape=[
            jax.ShapeDtypeStruct((B, D), jnp.float32),
            jax.ShapeDtypeStruct((B, G, D), jnp.float32),
        ],
        compiler_params=pltpu.CompilerParams(
            dimension_semantics=("arbitrary",),
        ),
    )(x_dict, emb)
    return (feat, gene_emb)
